# baseline (speedup 1.0000x reference)
import functools

import jax
import jax.numpy as jnp
from jax.experimental import pallas as pl
from jax.experimental.pallas import tpu as pltpu

_EPS = 1e-05
_NE = 22
_T = 1000
_K1 = 32
_NH = 4
_HD = 8
_F2 = 32
_NF = 40
_NEDGE = 110
_E4 = _NEDGE * _NH
_A4 = _NE * _NH
_K3 = 16
_T8 = 125
_T4 = 31
_NB = 2
_TW = _NB * _T
_XW = 1032

_HI = jax.lax.Precision.HIGHEST
_DF = jax.lax.Precision.DEFAULT


def _mm(a, b, precision=_DF):
    return jax.lax.dot_general(a, b, (((1,), (0,)), ((), ())),
                               preferred_element_type=jnp.float32,
                               precision=precision)


def _lrelu(v):
    return jnp.where(v > 0, v, 0.2 * v)


def _fwd_kernel(x_ref, g2_ref, b0_ref, ei_ref, eit_ref, rh_ref, q_ref,
                sg_ref, bg_ref, w3_ref, s3_ref, b3_ref, p8_ref, p4_ref,
                o_ref):
    f32 = jnp.float32
    i32 = jnp.int32
    xp = x_ref[0]

    Pa = jnp.concatenate(
        [jnp.concatenate([xp[:11, _XW * b + k:_XW * b + k + _T]
                          for b in range(_NB)], axis=1)
         for k in range(_K1)], axis=0)
    Pb = jnp.concatenate(
        [jnp.concatenate([xp[11:, _XW * b + k:_XW * b + k + _T]
                          for b in range(_NB)], axis=1)
         for k in range(_K1)], axis=0)
    ga = g2_ref[...]
    NF2 = jnp.concatenate([_mm(ga, Pa), _mm(ga, Pb)], axis=0) + b0_ref[...]
    nf_feat = jnp.concatenate(
        [NF2[:_F2 * 11], NF2[440:440 + _F2 * 11]], axis=0)
    ash_all = jnp.concatenate(
        [NF2[352:396], NF2[792:836]], axis=0)
    adh_all = jnp.concatenate(
        [NF2[396:440], NF2[836:880]], axis=0)
    asd_all = jnp.concatenate([ash_all, adh_all], axis=0)

    ei = ei_ref[...]
    src_r = ei[0:1, :_NEDGE]
    dst_r = ei[1:2, :_NEDGE]
    eit = eit_ref[...]
    src_c = eit[:_NEDGE, 0:1]
    dst_c = eit[:_NEDGE, 1:2]
    src_c4 = jnp.concatenate([src_c] * _NH, axis=0)
    dst_c4 = jnp.concatenate([dst_c] * _NH, axis=0)
    src_r4 = jnp.concatenate([src_r] * _NH, axis=1)
    dst_r4 = jnp.concatenate([dst_r] * _NH, axis=1)

    hrow_g = jax.lax.broadcasted_iota(i32, (_E4, _A4), 0) // _NEDGE
    cio_g = jax.lax.broadcasted_iota(i32, (_E4, _A4), 1)
    hcol_g = (cio_g % 44) // 11
    ncol_g = (cio_g // 44) * 11 + cio_g % 11
    same_h_g = hrow_g == hcol_g
    Ssrc4 = (same_h_g & (ncol_g == src_c4)).astype(f32)
    Sdst4 = (same_h_g & (ncol_g == dst_c4)).astype(f32)

    rio_s = jax.lax.broadcasted_iota(i32, (_A4, _E4), 0)
    hrow_s = (rio_s % 44) // 11
    nrow_s = (rio_s // 44) * 11 + rio_s % 11
    hcol_s = jax.lax.broadcasted_iota(i32, (_A4, _E4), 1) // _NEDGE
    same_h_s = hrow_s == hcol_s
    Dsum4 = (same_h_s & (nrow_s == dst_r4)).astype(f32)
    Ssum4 = (same_h_s & (nrow_s == src_r4)).astype(f32)

    SS = jnp.concatenate([Ssrc4, Sdst4], axis=1)
    E = _lrelu(_mm(SS, asd_all))

    c_parts = []
    for h in range(_NH):
        r0 = 11 * h
        amax_h = jnp.maximum(
            jnp.max(ash_all[r0:r0 + 11], axis=0, keepdims=True),
            jnp.max(ash_all[44 + r0:44 + r0 + 11], axis=0, keepdims=True))
        dmax_h = jnp.maximum(
            jnp.max(adh_all[r0:r0 + 11], axis=0, keepdims=True),
            jnp.max(adh_all[44 + r0:44 + r0 + 11], axis=0, keepdims=True))
        c_parts.append(jnp.broadcast_to(_lrelu(amax_h + dmax_h),
                                        (_NEDGE, _TW)))
    c440 = jnp.concatenate(c_parts, axis=0)
    ee = jnp.exp(E - c440)
    denom = _mm(Dsum4, ee)
    inv = 1.0 / (denom + 1e-16)
    iedge = _mm(Sdst4, inv)
    pn = ee * iedge
    wcol_all = _mm(Ssum4, pn)

    wrep = _mm(rh_ref[...], wcol_all)
    gm = _mm(q_ref[...], nf_feat * wrep)

    z = gm * sg_ref[...] + bg_ref[...]
    z = jnp.where(z > 0, z, jnp.exp(jnp.minimum(z, 0.0)) - 1.0)
    z8 = _mm(z, p8_ref[...])

    zpad = jnp.concatenate(
        [jnp.zeros((_F2, 7), f32), z8[:, :_T8], jnp.zeros((_F2, 15), f32),
         z8[:, _T8:], jnp.zeros((_F2, 8), f32)], axis=1)
    zst = jnp.concatenate([zpad[:, k:k + 265] for k in range(_K3)], axis=0)
    c3 = _mm(w3_ref[...], zst)
    c3 = c3 * s3_ref[...] + b3_ref[...]
    c3 = jnp.where(c3 > 0, c3, jnp.exp(jnp.minimum(c3, 0.0)) - 1.0)
    o2 = _mm(c3, p4_ref[...])
    o_ref[0] = o2[:, :_T4]
    o_ref[1] = o2[:, _T4:]


@functools.partial(jax.jit, static_argnames=("interpret",))
def _run(x, conv1_w, gamma1, beta1, W_gat, att_src, att_dst, bias_gat,
         gamma_g, beta_g, conv3_w, gamma3, beta3, edge_index,
         interpret=False):
    f32 = jnp.float32
    B = x.shape[0]

    inv_sqrt = 1.0 / jnp.sqrt(1.0 + _EPS)
    scale1 = gamma1 * inv_sqrt
    w1s = conv1_w[:, 0, 0, :] * scale1[:, None]
    Wf = w1s.T @ W_gat
    b0 = beta1 @ W_gat
    eye4 = jnp.eye(_NH, dtype=f32)
    as_mat = (att_src[:, :, None] * eye4[:, None, :]).reshape(_F2, _NH)
    ad_mat = (att_dst[:, :, None] * eye4[:, None, :]).reshape(_F2, _NH)
    Wf_ext = jnp.concatenate([Wf, Wf @ as_mat, Wf @ ad_mat], axis=1)
    b0_ext = jnp.concatenate([b0, b0 @ as_mat, b0 @ ad_mat])
    b0col = b0_ext[(jnp.arange(2 * 440) % 440) // 11][:, None]

    eye11 = jnp.eye(11, dtype=f32)
    G2 = (Wf_ext.T[:, None, :, None] * eye11[None, :, None, :]
          ).reshape(_NF * 11, _K1 * 11)

    rf = jnp.arange(_F2 * _NE)
    half_f, j_f, el_f = rf // 352, (rf % 352) // 11, rf % 11
    cc = jnp.arange(_NH * _NE)
    half_c, h_c, el_c = cc // 44, (cc % 44) // 11, cc % 11
    Rh = ((half_f[:, None] == half_c[None, :])
          & (el_f[:, None] == el_c[None, :])
          & ((j_f[:, None] // _HD) == h_c[None, :])).astype(f32)
    Q = (jnp.arange(_F2)[:, None] == j_f[None, :]).astype(f32)

    sg = gamma_g * inv_sqrt
    sg2 = (sg / f32(_NE))[:, None]
    bg2 = (bias_gat * sg + beta_g)[:, None]

    W3flat = jnp.transpose(conv3_w[:, :, 0, :], (0, 2, 1)).reshape(
        _F2, _K3 * _F2)
    s3 = (gamma3 * inv_sqrt)[:, None]
    b3 = beta3[:, None]

    p8m = ((jnp.arange(_TW)[:, None] // 8) == jnp.arange(_NB * _T8)[None, :]
           ).astype(f32) / 8.0
    tp = jnp.arange(265)
    tt = tp % 140
    cglob = (tp // 140) * _T4 + tt // 4
    p4m = ((cglob[:, None] == jnp.arange(_NB * _T4)[None, :])
           & (tt < 124)[:, None]).astype(f32) / 4.0

    ei_pad = jnp.zeros((8, 128), jnp.int32).at[:2, :_NEDGE].set(edge_index)
    eit_pad = jnp.zeros((128, 8), jnp.int32).at[:_NEDGE, :2].set(
        edge_index.T)
    xpad = jnp.pad(x[:, 0], ((0, 0), (0, 0), (15, 17)))
    xg = xpad.reshape(B // _NB, _NB, _NE, _XW).transpose(0, 2, 1, 3
        ).reshape(B // _NB, _NE, _NB * _XW)

    full = lambda a: pl.BlockSpec(a.shape, lambda b: (0,) * a.ndim)
    out = pl.pallas_call(
        _fwd_kernel,
        grid=(B // _NB,),
        in_specs=[
            pl.BlockSpec((1, _NE, _NB * _XW), lambda b: (b, 0, 0)),
            full(G2), full(b0col), full(ei_pad), full(eit_pad),
            full(Rh), full(Q), full(sg2), full(bg2),
            full(W3flat), full(s3), full(b3), full(p8m), full(p4m),
        ],
        out_specs=pl.BlockSpec((_NB, _F2, _T4), lambda b: (b, 0, 0)),
        out_shape=jax.ShapeDtypeStruct((B, _F2, _T4), f32),
        compiler_params=pltpu.CompilerParams(
            dimension_semantics=("parallel",)),
        interpret=interpret,
    )(xg, G2, b0col, ei_pad, eit_pad, Rh, Q, sg2, bg2, W3flat, s3, b3,
      p8m, p4m)
    return out[:, :, None, :]


def kernel(x, conv1_w, gamma1, beta1, W_gat, att_src, att_dst, bias_gat,
           gamma_g, beta_g, conv3_w, gamma3, beta3, edge_index):
    return _run(x, conv1_w, gamma1, beta1, W_gat, att_src, att_dst, bias_gat,
                gamma_g, beta_g, conv3_w, gamma3, beta3, edge_index)


# speedup vs baseline: 1.1954x; 1.1954x over previous
import functools

import jax
import jax.numpy as jnp
from jax.experimental import pallas as pl
from jax.experimental.pallas import tpu as pltpu

_EPS = 1e-05
_NE = 22
_T = 1000
_K1 = 32
_NH = 4
_HD = 8
_F2 = 32
_NF = 40
_NEDGE = 110
_E4 = _NEDGE * _NH
_A4 = _NE * _NH
_K3 = 16
_T8 = 125
_T4 = 31
_NB = 2
_TW = _NB * _T
_XW = 1032

_HI = jax.lax.Precision.HIGHEST
_DF = jax.lax.Precision.DEFAULT


def _mm(a, b, precision=_DF):
    return jax.lax.dot_general(a, b, (((1,), (0,)), ((), ())),
                               preferred_element_type=jnp.float32,
                               precision=precision)


def _lrelu(v):
    return jnp.where(v > 0, v, 0.2 * v)


def _fwd_kernel(x_ref, g2_ref, b0_ref, ei_ref, eit_ref, rh_ref, q_ref,
                sg_ref, bg_ref, w3_ref, s3_ref, b3_ref, p8_ref, p4_ref,
                o_ref):
    f32 = jnp.float32
    i32 = jnp.int32
    xp = x_ref[0]

    P = jnp.concatenate(
        [jnp.concatenate([xp[:, _XW * b + k:_XW * b + k + _T]
                          for b in range(_NB)], axis=1)
         for k in range(_K1)], axis=0)
    NF2 = _mm(g2_ref[...], P) + b0_ref[...]
    nf_feat = NF2[:_F2 * _NE]
    ash_all = NF2[_F2 * _NE:(_F2 + _NH) * _NE]
    adh_all = NF2[(_F2 + _NH) * _NE:]

    ei = ei_ref[...]
    src_r = ei[0:1, :_NEDGE]
    dst_r = ei[1:2, :_NEDGE]
    eit = eit_ref[...]
    src_c = eit[:_NEDGE, 0:1]
    dst_c = eit[:_NEDGE, 1:2]
    src_c4 = jnp.concatenate([src_c] * _NH, axis=0)
    dst_c4 = jnp.concatenate([dst_c] * _NH, axis=0)
    src_r4 = jnp.concatenate([src_r] * _NH, axis=1)
    dst_r4 = jnp.concatenate([dst_r] * _NH, axis=1)

    hrow_g = jax.lax.broadcasted_iota(i32, (_E4, _A4), 0) // _NEDGE
    hcol_g = jax.lax.broadcasted_iota(i32, (_E4, _A4), 1) // _NE
    ncol_g = jax.lax.broadcasted_iota(i32, (_E4, _A4), 1) % _NE
    same_h_g = hrow_g == hcol_g
    Ssrc4 = (same_h_g & (ncol_g == src_c4)).astype(f32)
    Sdst4 = (same_h_g & (ncol_g == dst_c4)).astype(f32)

    hrow_s = jax.lax.broadcasted_iota(i32, (_A4, _E4), 0) // _NE
    nrow_s = jax.lax.broadcasted_iota(i32, (_A4, _E4), 0) % _NE
    hcol_s = jax.lax.broadcasted_iota(i32, (_A4, _E4), 1) // _NEDGE
    same_h_s = hrow_s == hcol_s
    Dsum4 = (same_h_s & (nrow_s == dst_r4)).astype(f32)
    Ssum4 = (same_h_s & (nrow_s == src_r4)).astype(f32)

    SS = jnp.concatenate([Ssrc4, Sdst4], axis=1)
    E = _lrelu(_mm(SS, NF2[_F2 * _NE:]))

    c_parts = []
    for h in range(_NH):
        amax_h = jnp.max(ash_all[_NE * h:_NE * (h + 1)], axis=0,
                         keepdims=True)
        dmax_h = jnp.max(adh_all[_NE * h:_NE * (h + 1)], axis=0,
                         keepdims=True)
        c_parts.append(jnp.broadcast_to(_lrelu(amax_h + dmax_h),
                                        (_NEDGE, _TW)))
    c440 = jnp.concatenate(c_parts, axis=0)
    ee = jnp.exp(E - c440)
    denom = _mm(Dsum4, ee)
    inv = 1.0 / (denom + 1e-16)
    iedge = _mm(Sdst4, inv)
    pn = ee * iedge
    wcol_all = _mm(Ssum4, pn)

    wrep = _mm(rh_ref[...], wcol_all)
    gm = _mm(q_ref[...], nf_feat * wrep)

    z = gm * sg_ref[...] + bg_ref[...]
    z = jnp.where(z > 0, z, jnp.exp(jnp.minimum(z, 0.0)) - 1.0)
    z8 = _mm(z, p8_ref[...])

    zpad = jnp.concatenate(
        [jnp.zeros((_F2, 7), f32), z8[:, :_T8], jnp.zeros((_F2, 15), f32),
         z8[:, _T8:], jnp.zeros((_F2, 8), f32)], axis=1)
    zst = jnp.concatenate([zpad[:, k:k + 265] for k in range(_K3)], axis=0)
    c3 = _mm(w3_ref[...], zst)
    c3 = c3 * s3_ref[...] + b3_ref[...]
    c3 = jnp.where(c3 > 0, c3, jnp.exp(jnp.minimum(c3, 0.0)) - 1.0)
    o2 = _mm(c3, p4_ref[...])
    o_ref[0] = o2[:, :_T4]
    o_ref[1] = o2[:, _T4:]


@functools.partial(jax.jit, static_argnames=("interpret",))
def _run(x, conv1_w, gamma1, beta1, W_gat, att_src, att_dst, bias_gat,
         gamma_g, beta_g, conv3_w, gamma3, beta3, edge_index,
         interpret=False):
    f32 = jnp.float32
    B = x.shape[0]

    inv_sqrt = 1.0 / jnp.sqrt(1.0 + _EPS)
    scale1 = gamma1 * inv_sqrt
    w1s = conv1_w[:, 0, 0, :] * scale1[:, None]
    Wf = w1s.T @ W_gat
    b0 = beta1 @ W_gat
    eye4 = jnp.eye(_NH, dtype=f32)
    as_mat = (att_src[:, :, None] * eye4[:, None, :]).reshape(_F2, _NH)
    ad_mat = (att_dst[:, :, None] * eye4[:, None, :]).reshape(_F2, _NH)
    Wf_ext = jnp.concatenate([Wf, Wf @ as_mat, Wf @ ad_mat], axis=1)
    b0_ext = jnp.concatenate([b0, b0 @ as_mat, b0 @ ad_mat])
    b0col = jnp.repeat(b0_ext, _NE)[:, None]

    eye22 = jnp.eye(_NE, dtype=f32)
    G2 = (Wf_ext.T[:, None, :, None] * eye22[None, :, None, :]
          ).reshape(_NF * _NE, _K1 * _NE)

    jh = jnp.arange(_F2)
    hsel = ((jh[:, None] // _HD) == jnp.arange(_NH)[None, :]).astype(f32)
    Rh = (hsel[:, None, :, None] * eye22[None, :, None, :]
          ).reshape(_F2 * _NE, _NH * _NE)
    qsel = (jnp.arange(_F2)[:, None] == jnp.arange(_F2)[None, :]).astype(f32)
    Q = jnp.broadcast_to(qsel[:, :, None], (_F2, _F2, _NE)
                         ).reshape(_F2, _F2 * _NE)

    sg = gamma_g * inv_sqrt
    sg2 = (sg / f32(_NE))[:, None]
    bg2 = (bias_gat * sg + beta_g)[:, None]

    W3flat = jnp.transpose(conv3_w[:, :, 0, :], (0, 2, 1)).reshape(
        _F2, _K3 * _F2)
    s3 = (gamma3 * inv_sqrt)[:, None]
    b3 = beta3[:, None]

    p8m = ((jnp.arange(_TW)[:, None] // 8) == jnp.arange(_NB * _T8)[None, :]
           ).astype(f32) / 8.0
    tp = jnp.arange(265)
    tt = tp % 140
    cglob = (tp // 140) * _T4 + tt // 4
    p4m = ((cglob[:, None] == jnp.arange(_NB * _T4)[None, :])
           & (tt < 124)[:, None]).astype(f32) / 4.0

    ei_pad = jnp.zeros((8, 128), jnp.int32).at[:2, :_NEDGE].set(edge_index)
    eit_pad = jnp.zeros((128, 8), jnp.int32).at[:_NEDGE, :2].set(
        edge_index.T)
    xpad = jnp.pad(x[:, 0], ((0, 0), (0, 0), (15, 17)))
    xg = xpad.reshape(B // _NB, _NB, _NE, _XW).transpose(0, 2, 1, 3
        ).reshape(B // _NB, _NE, _NB * _XW)

    full = lambda a: pl.BlockSpec(a.shape, lambda b: (0,) * a.ndim)
    out = pl.pallas_call(
        _fwd_kernel,
        grid=(B // _NB,),
        in_specs=[
            pl.BlockSpec((1, _NE, _NB * _XW), lambda b: (b, 0, 0)),
            full(G2), full(b0col), full(ei_pad), full(eit_pad),
            full(Rh), full(Q), full(sg2), full(bg2),
            full(W3flat), full(s3), full(b3), full(p8m), full(p4m),
        ],
        out_specs=pl.BlockSpec((_NB, _F2, _T4), lambda b: (b, 0, 0)),
        out_shape=jax.ShapeDtypeStruct((B, _F2, _T4), f32),
        compiler_params=pltpu.CompilerParams(
            dimension_semantics=("parallel",)),
        interpret=interpret,
    )(xg, G2, b0col, ei_pad, eit_pad, Rh, Q, sg2, bg2, W3flat, s3, b3,
      p8m, p4m)
    return out[:, :, None, :]


def kernel(x, conv1_w, gamma1, beta1, W_gat, att_src, att_dst, bias_gat,
           gamma_g, beta_g, conv3_w, gamma3, beta3, edge_index):
    return _run(x, conv1_w, gamma1, beta1, W_gat, att_src, att_dst, bias_gat,
                gamma_g, beta_g, conv3_w, gamma3, beta3, edge_index)


# speedup vs baseline: 1.1967x; 1.0011x over previous
import jax
import jax.numpy as jnp
from jax.experimental import pallas as pl
from jax.experimental.pallas import tpu as pltpu

_EPS = 1e-05
_NE = 22
_T = 1000
_K1 = 32
_NH = 4
_HD = 8
_F2 = 32
_NF = 40
_NEDGE = 110
_E4 = _NEDGE * _NH
_A4 = _NE * _NH
_K3 = 16
_T8 = 125
_T4 = 31
_NB = 2
_TW = _NB * _T
_XW = 1032

_HI = jax.lax.Precision.HIGHEST
_DF = jax.lax.Precision.DEFAULT


def _mm(a, b, precision=_DF):
    return jax.lax.dot_general(a, b, (((1,), (0,)), ((), ())),
                               preferred_element_type=jnp.float32,
                               precision=precision)


def _lrelu(v):
    return jnp.where(v > 0, v, 0.2 * v)


def _fwd_kernel(x_ref, g2_ref, b0_ref, ei_ref, eit_ref, rh_ref, q_ref,
                sg_ref, bg_ref, w3_ref, s3_ref, b3_ref, p8_ref, p4_ref,
                o_ref):
    f32 = jnp.float32
    i32 = jnp.int32
    xp = x_ref[0]

    P = jnp.concatenate(
        [jnp.concatenate([xp[:, _XW * b + k:_XW * b + k + _T]
                          for b in range(_NB)], axis=1)
         for k in range(_K1)], axis=0)
    NF2 = _mm(g2_ref[...], P) + b0_ref[...]
    nf_feat = NF2[:_F2 * _NE]
    ash_all = NF2[_F2 * _NE:(_F2 + _NH) * _NE]
    adh_all = NF2[(_F2 + _NH) * _NE:]

    ei = ei_ref[...]
    src_r = ei[0:1, :_NEDGE]
    dst_r = ei[1:2, :_NEDGE]
    eit = eit_ref[...]
    src_c = eit[:_NEDGE, 0:1]
    dst_c = eit[:_NEDGE, 1:2]
    src_c4 = jnp.concatenate([src_c] * _NH, axis=0)
    dst_c4 = jnp.concatenate([dst_c] * _NH, axis=0)
    src_r4 = jnp.concatenate([src_r] * _NH, axis=1)
    dst_r4 = jnp.concatenate([dst_r] * _NH, axis=1)

    hrow_g = jax.lax.broadcasted_iota(i32, (_E4, _A4), 0) // _NEDGE
    hcol_g = jax.lax.broadcasted_iota(i32, (_E4, _A4), 1) // _NE
    ncol_g = jax.lax.broadcasted_iota(i32, (_E4, _A4), 1) % _NE
    same_h_g = hrow_g == hcol_g
    Ssrc4 = (same_h_g & (ncol_g == src_c4)).astype(f32)
    Sdst4 = (same_h_g & (ncol_g == dst_c4)).astype(f32)

    hrow_s = jax.lax.broadcasted_iota(i32, (_A4, _E4), 0) // _NE
    nrow_s = jax.lax.broadcasted_iota(i32, (_A4, _E4), 0) % _NE
    hcol_s = jax.lax.broadcasted_iota(i32, (_A4, _E4), 1) // _NEDGE
    same_h_s = hrow_s == hcol_s
    Dsum4 = (same_h_s & (nrow_s == dst_r4)).astype(f32)
    Ssum4 = (same_h_s & (nrow_s == src_r4)).astype(f32)

    SS = jnp.concatenate([Ssrc4, Sdst4], axis=1)
    E = _lrelu(_mm(SS, NF2[_F2 * _NE:]))

    c_parts = []
    for h in range(_NH):
        amax_h = jnp.max(ash_all[_NE * h:_NE * (h + 1)], axis=0,
                         keepdims=True)
        dmax_h = jnp.max(adh_all[_NE * h:_NE * (h + 1)], axis=0,
                         keepdims=True)
        c_parts.append(jnp.broadcast_to(_lrelu(amax_h + dmax_h),
                                        (_NEDGE, _TW)))
    c440 = jnp.concatenate(c_parts, axis=0)
    ee = jnp.exp(E - c440)
    denom = _mm(Dsum4, ee)
    inv = 1.0 / (denom + 1e-16)
    iedge = _mm(Sdst4, inv)
    pn = ee * iedge
    wcol_all = _mm(Ssum4, pn)

    wrep = _mm(rh_ref[...], wcol_all)
    gm = _mm(q_ref[...], nf_feat * wrep)

    z = gm * sg_ref[...] + bg_ref[...]
    z = jnp.where(z > 0, z, jnp.exp(jnp.minimum(z, 0.0)) - 1.0)
    z8 = _mm(z, p8_ref[...])

    zpad = jnp.concatenate(
        [jnp.zeros((_F2, 7), f32), z8[:, :_T8], jnp.zeros((_F2, 15), f32),
         z8[:, _T8:], jnp.zeros((_F2, 8), f32)], axis=1)
    zst = jnp.concatenate([zpad[:, k:k + 265] for k in range(_K3)], axis=0)
    c3 = _mm(w3_ref[...], zst)
    c3 = c3 * s3_ref[...] + b3_ref[...]
    c3 = jnp.where(c3 > 0, c3, jnp.exp(jnp.minimum(c3, 0.0)) - 1.0)
    o2 = _mm(c3, p4_ref[...])
    o_ref[0] = o2[:, :_T4]
    o_ref[1] = o2[:, _T4:]


@jax.jit
def _run(x, conv1_w, gamma1, beta1, W_gat, att_src, att_dst, bias_gat,
         gamma_g, beta_g, conv3_w, gamma3, beta3, edge_index):
    f32 = jnp.float32
    B = x.shape[0]

    inv_sqrt = 1.0 / jnp.sqrt(1.0 + _EPS)
    scale1 = gamma1 * inv_sqrt
    w1s = conv1_w[:, 0, 0, :] * scale1[:, None]
    Wf = w1s.T @ W_gat
    b0 = beta1 @ W_gat
    eye4 = jnp.eye(_NH, dtype=f32)
    as_mat = (att_src[:, :, None] * eye4[:, None, :]).reshape(_F2, _NH)
    ad_mat = (att_dst[:, :, None] * eye4[:, None, :]).reshape(_F2, _NH)
    Wf_ext = jnp.concatenate([Wf, Wf @ as_mat, Wf @ ad_mat], axis=1)
    b0_ext = jnp.concatenate([b0, b0 @ as_mat, b0 @ ad_mat])
    b0col = jnp.repeat(b0_ext, _NE)[:, None]

    eye22 = jnp.eye(_NE, dtype=f32)
    G2 = (Wf_ext.T[:, None, :, None] * eye22[None, :, None, :]
          ).reshape(_NF * _NE, _K1 * _NE)

    jh = jnp.arange(_F2)
    hsel = ((jh[:, None] // _HD) == jnp.arange(_NH)[None, :]).astype(f32)
    Rh = (hsel[:, None, :, None] * eye22[None, :, None, :]
          ).reshape(_F2 * _NE, _NH * _NE)
    qsel = (jnp.arange(_F2)[:, None] == jnp.arange(_F2)[None, :]).astype(f32)
    Q = jnp.broadcast_to(qsel[:, :, None], (_F2, _F2, _NE)
                         ).reshape(_F2, _F2 * _NE)

    sg = gamma_g * inv_sqrt
    sg2 = (sg / f32(_NE))[:, None]
    bg2 = (bias_gat * sg + beta_g)[:, None]

    W3flat = jnp.transpose(conv3_w[:, :, 0, :], (0, 2, 1)).reshape(
        _F2, _K3 * _F2)
    s3 = (gamma3 * inv_sqrt)[:, None]
    b3 = beta3[:, None]

    p8m = ((jnp.arange(_TW)[:, None] // 8) == jnp.arange(_NB * _T8)[None, :]
           ).astype(f32) / 8.0
    tp = jnp.arange(265)
    tt = tp % 140
    cglob = (tp // 140) * _T4 + tt // 4
    p4m = ((cglob[:, None] == jnp.arange(_NB * _T4)[None, :])
           & (tt < 124)[:, None]).astype(f32) / 4.0

    ei_pad = jnp.zeros((8, 128), jnp.int32).at[:2, :_NEDGE].set(edge_index)
    eit_pad = jnp.zeros((128, 8), jnp.int32).at[:_NEDGE, :2].set(
        edge_index.T)
    xpad = jnp.pad(x[:, 0], ((0, 0), (0, 0), (15, 17)))
    xg = xpad.reshape(B // _NB, _NB, _NE, _XW).transpose(0, 2, 1, 3
        ).reshape(B // _NB, _NE, _NB * _XW)

    full = lambda a: pl.BlockSpec(a.shape, lambda b: (0,) * a.ndim)
    out = pl.pallas_call(
        _fwd_kernel,
        grid=(B // _NB,),
        in_specs=[
            pl.BlockSpec((1, _NE, _NB * _XW), lambda b: (b, 0, 0)),
            full(G2), full(b0col), full(ei_pad), full(eit_pad),
            full(Rh), full(Q), full(sg2), full(bg2),
            full(W3flat), full(s3), full(b3), full(p8m), full(p4m),
        ],
        out_specs=pl.BlockSpec((_NB, _F2, _T4), lambda b: (b, 0, 0)),
        out_shape=jax.ShapeDtypeStruct((B, _F2, _T4), f32),
        compiler_params=pltpu.CompilerParams(
            dimension_semantics=("parallel",)),
    )(xg, G2, b0col, ei_pad, eit_pad, Rh, Q, sg2, bg2, W3flat, s3, b3,
      p8m, p4m)
    return out[:, :, None, :]


def kernel(x, conv1_w, gamma1, beta1, W_gat, att_src, att_dst, bias_gat,
           gamma_g, beta_g, conv3_w, gamma3, beta3, edge_index):
    return _run(x, conv1_w, gamma1, beta1, W_gat, att_src, att_dst, bias_gat,
                gamma_g, beta_g, conv3_w, gamma3, beta3, edge_index)
